# Initial kernel scaffold; baseline (speedup 1.0000x reference)
import jax
import jax.numpy as jnp
from jax.experimental import pallas as pl


def kernel(x, edge_index, W1, b1, W2, b2, W3, b3, Wc1, bc1, Wc2, bc2):
    raise NotImplementedError("write your pallas kernel here")


# speedup vs baseline: 4.6086x; 4.6086x over previous
import functools
import jax
import jax.numpy as jnp
from jax import lax
from jax.experimental import pallas as pl
from jax.experimental.pallas import tpu as pltpu
from jax.experimental.pallas import tpu_sc as plsc

_NC = 2
_NS = 16
_NW = _NC * _NS


def _make_deg(n_pad, e, k):
    epw = e // _NW
    nchunks = epw // k
    rpt = n_pad // _NS
    mesh = plsc.VectorSubcoreMesh(core_axis_name="c", subcore_axis_name="s")

    @functools.partial(
        pl.kernel,
        out_type=jax.ShapeDtypeStruct((_NC * n_pad,), jnp.float32),
        mesh=mesh,
        scratch_types=[
            pltpu.VMEM((k,), jnp.int32),
            pltpu.VMEM((k,), jnp.float32),
            pltpu.VMEM((rpt,), jnp.float32),
            pltpu.VMEM_SHARED((n_pad,), jnp.float32),
        ],
    )
    def deg_kernel(dst_hbm, out_hbm, didx, ones, zbuf, acc):
        c = lax.axis_index("c")
        s = lax.axis_index("s")
        wid = c * _NS + s

        def fill(i, _):
            ones[pl.ds(i * 16, 16)] = jnp.full((16,), 1.0, jnp.float32)
            return 0
        lax.fori_loop(0, k // 16, fill, 0)

        def zfill(i, _):
            zbuf[pl.ds(i * 16, 16)] = jnp.zeros((16,), jnp.float32)
            return 0
        lax.fori_loop(0, rpt // 16, zfill, 0)
        pltpu.sync_copy(zbuf, acc.at[pl.ds(s * rpt, rpt)])
        plsc.subcore_barrier()

        base = wid * epw

        def chunk(i, _):
            off = pl.multiple_of(base + i * k, 8)
            pltpu.sync_copy(dst_hbm.at[pl.ds(off, k)], didx)
            pltpu.sync_copy(ones, acc.at[didx], add=True)
            return 0
        lax.fori_loop(0, nchunks, chunk, 0)
        plsc.subcore_barrier()

        o0 = pl.multiple_of(c * n_pad + s * rpt, 8)
        pltpu.sync_copy(acc.at[pl.ds(s * rpt, rpt)], zbuf)
        pltpu.sync_copy(zbuf, out_hbm.at[pl.ds(o0, rpt)])

    return deg_kernel


def _make_agg(n_acc, e, d, k):
    n = n_acc
    epw = e // _NW
    nchunks = epw // k
    rpt = n // _NS
    mesh = plsc.VectorSubcoreMesh(core_axis_name="c", subcore_axis_name="s")

    @functools.partial(
        pl.kernel,
        out_type=jax.ShapeDtypeStruct((_NC, n, d), jnp.float32),
        mesh=mesh,
        scratch_types=[
            pltpu.VMEM((k,), jnp.int32),
            pltpu.VMEM((k,), jnp.int32),
            pltpu.VMEM((k, d), jnp.float32),
            pltpu.VMEM_SHARED((n, d), jnp.float32),
            pltpu.SemaphoreType.DMA,
        ],
    )
    def agg_kernel(y_hbm, src_hbm, dst_hbm, out_hbm, sidx, didx, rows, acc, sem):
        c = lax.axis_index("c")
        s = lax.axis_index("s")
        wid = c * _NS + s

        def zrow(i, _):
            def zcol(j, _):
                rows[i, pl.ds(j * 16, 16)] = jnp.zeros((16,), jnp.float32)
                return 0
            return lax.fori_loop(0, d // 16, zcol, 0)
        lax.fori_loop(0, k, zrow, 0)

        nfull = rpt // k
        rem = rpt - nfull * k
        r0 = s * rpt

        def zcopy(i, _):
            pltpu.sync_copy(rows, acc.at[pl.ds(r0 + i * k, k)])
            return 0
        lax.fori_loop(0, nfull, zcopy, 0)
        if rem:
            pltpu.sync_copy(rows.at[pl.ds(0, rem)],
                            acc.at[pl.ds(r0 + nfull * k, rem)])
        plsc.subcore_barrier()

        base = wid * epw

        def chunk(i, _):
            off = pl.multiple_of(base + i * k, 8)
            pltpu.sync_copy(src_hbm.at[pl.ds(off, k)], sidx)
            pltpu.sync_copy(dst_hbm.at[pl.ds(off, k)], didx)
            pltpu.async_copy(y_hbm.at[sidx], rows, sem).wait()
            pltpu.sync_copy(rows, acc.at[didx], add=True)
            return 0
        lax.fori_loop(0, nchunks, chunk, 0)
        plsc.subcore_barrier()

        def ocopy(i, _):
            pltpu.sync_copy(acc.at[pl.ds(r0 + i * k, k)], rows)
            pltpu.sync_copy(rows, out_hbm.at[c, pl.ds(r0 + i * k, k)])
            return 0
        lax.fori_loop(0, nfull, ocopy, 0)
        if rem:
            pltpu.sync_copy(acc.at[pl.ds(r0 + nfull * k, rem)],
                            rows.at[pl.ds(0, rem)])
            pltpu.sync_copy(rows.at[pl.ds(0, rem)],
                            out_hbm.at[c, pl.ds(r0 + nfull * k, rem)])

    return agg_kernel


def _t1_body(x_ref, d0_ref, d1_ref, w0_ref, wt_ref, out_ref):
    deg = d0_ref[...] + d1_ref[...]
    out_ref[...] = (
        jnp.dot(x_ref[...], wt_ref[...], preferred_element_type=jnp.float32,
                precision=lax.Precision.HIGHEST)
        + deg * w0_ref[...]
    )


def _mid_body(t_ref, a0_ref, a1_ref, d0_ref, d1_ref, b_ref, wt_ref, out_ref):
    deg = d0_ref[...] + d1_ref[...]
    mean = (a0_ref[...] + a1_ref[...]) / jnp.maximum(deg, 1.0)
    hup = jnp.where(deg > 0.0, mean, t_ref[...])
    h = jnp.maximum(hup + b_ref[...], 0.0)
    out_ref[...] = jnp.dot(h, wt_ref[...], preferred_element_type=jnp.float32,
                           precision=lax.Precision.HIGHEST)


def _final_body(n, ngrid, t_ref, a0_ref, a1_ref, d0_ref, d1_ref, b_ref,
                wc1t_ref, bc1_ref, wc2t_ref, bc2_ref, out_ref, acc_ref):
    i = pl.program_id(0)
    deg = d0_ref[...] + d1_ref[...]
    mean = (a0_ref[...] + a1_ref[...]) / jnp.maximum(deg, 1.0)
    hup = jnp.where(deg > 0.0, mean, t_ref[...])
    h = jnp.maximum(hup + b_ref[...], 0.0)
    part = jnp.sum(h, axis=0, keepdims=True)

    @pl.when(i == 0)
    def _():
        acc_ref[...] = part

    @pl.when(i > 0)
    def _():
        acc_ref[...] += part

    @pl.when(i == ngrid - 1)
    def _():
        hg = acc_ref[...] / float(n)
        hg = jnp.dot(hg, wc1t_ref[...], preferred_element_type=jnp.float32,
                     precision=lax.Precision.HIGHEST) + bc1_ref[...]
        hg = jnp.dot(hg, wc1t_ref[...], preferred_element_type=jnp.float32,
                     precision=lax.Precision.HIGHEST) + bc1_ref[...]
        out_ref[...] = jnp.dot(hg, wc2t_ref[...],
                               preferred_element_type=jnp.float32,
                               precision=lax.Precision.HIGHEST) + bc2_ref[...]


def _row_spec(blk, d):
    return pl.BlockSpec((blk, d), lambda i: (i, 0))


def _full_spec(shape):
    return pl.BlockSpec(shape, lambda i: tuple(0 for _ in shape))


def kernel(x, edge_index, W1, b1, W2, b2, W3, b3, Wc1, bc1, Wc2, bc2):
    n, d = x.shape
    e = edge_index.shape[1]
    h = W1.shape[0]
    src = edge_index[0]
    dst = edge_index[1]

    n_pad = ((n + (8 * _NS) - 1) // (8 * _NS)) * (8 * _NS)
    k = 80

    deg_p = _make_deg(n_pad, e, k)(dst).reshape(_NC, n_pad)
    d0 = deg_p[0, :n].reshape(n, 1)
    d1 = deg_p[1, :n].reshape(n, 1)

    blk = 1000
    ngrid = n // blk
    row = functools.partial(_row_spec, blk)
    dspec = pl.BlockSpec((blk, 1), lambda i: (i, 0))

    t1 = pl.pallas_call(
        _t1_body,
        grid=(ngrid,),
        in_specs=[row(d), dspec, dspec, _full_spec((1, h)), _full_spec((d, h))],
        out_specs=row(h),
        out_shape=jax.ShapeDtypeStruct((n, h), jnp.float32),
    )(x, d0, d1, W1[:, 0].reshape(1, h), W1[:, 1:].T)

    agg = _make_agg(n_pad, e, h, k)

    mid = pl.pallas_call(
        _mid_body,
        grid=(ngrid,),
        in_specs=[row(h), row(h), row(h), dspec, dspec,
                  _full_spec((1, h)), _full_spec((h, h))],
        out_specs=row(h),
        out_shape=jax.ShapeDtypeStruct((n, h), jnp.float32),
    )

    a = agg(t1, src, dst)
    t2 = mid(t1, a[0], a[1], d0, d1, b1.reshape(1, h), W2.T)
    a = agg(t2, src, dst)
    t3 = mid(t2, a[0], a[1], d0, d1, b2.reshape(1, h), W3.T)
    a = agg(t3, src, dst)

    out = pl.pallas_call(
        functools.partial(_final_body, n, ngrid),
        grid=(ngrid,),
        in_specs=[row(h), row(h), row(h), dspec, dspec, _full_spec((1, h)),
                  _full_spec((h, h)), _full_spec((1, h)),
                  _full_spec((h, 1)), _full_spec((1, 1))],
        out_specs=_full_spec((1, 1)),
        out_shape=jax.ShapeDtypeStruct((1, 1), jnp.float32),
        scratch_shapes=[pltpu.VMEM((1, h), jnp.float32)],
    )(t3, a[0], a[1], d0, d1, b3.reshape(1, h),
      Wc1.T, bc1.reshape(1, h), Wc2.T, bc2.reshape(1, 1))

    return out
